# baseline (speedup 1.0000x reference)
import functools

import jax
import jax.numpy as jnp
from jax import lax
from jax.experimental import pallas as pl
from jax.experimental.pallas import tpu as pltpu
from jax.experimental.pallas import tpu_sc as plsc

_ROWS, _COLS = 16384, 512
_OUT_COLS = _COLS // 2
_NC, _NS, _L = 2, 16, 16
_NW = _NC * _NS
_RPW = _ROWS // _NW
_CHUNK = 64
_NBUF = 2
_NCHUNK = _RPW // _CHUNK
_NGROUP = _NCHUNK // _NBUF
_JPR = _OUT_COLS // _L


def _body(
    x_hbm, out_hbm,
    in0, in1, out0, out1,
    si0, si1, so0, so1,
):
    wid = lax.axis_index("s") * _NC + lax.axis_index("c")
    lane = lax.iota(jnp.int32, 16)
    row0 = wid * _RPW
    ins, outs = (in0, in1), (out0, out1)
    sis, sos = (si0, si1), (so0, so1)

    for b in range(_NBUF):
        pltpu.async_copy(
            x_hbm.at[pl.ds(row0 + b * _CHUNK, _CHUNK)], ins[b], sis[b]
        )

    def group(g, _):
        for b in range(_NBUF):
            c = _NBUF * g + b
            r0 = row0 + c * _CHUNK
            pltpu.make_async_copy(
                x_hbm.at[pl.ds(r0, _CHUNK)], ins[b], sis[b]
            ).wait()

            @pl.when(g > 0)
            def _():
                pltpu.make_async_copy(
                    outs[b], out_hbm.at[pl.ds(r0, _CHUNK)], sos[b]
                ).wait()

            ib, ob = ins[b], outs[b]

            @plsc.parallel_loop(0, _CHUNK, step=1, unroll=1)
            def _(r):
                rowv = jnp.full((16,), r, jnp.int32)
                for j in range(_JPR):
                    col = 2 * _L * j + 2 * lane
                    v = plsc.load_gather(ib, [rowv, col])
                    ob[r, pl.ds(_L * j, _L)] = v
            pltpu.async_copy(
                outs[b], out_hbm.at[pl.ds(r0, _CHUNK)], sos[b]
            )

            @pl.when(g < _NGROUP - 1)
            def _():
                pltpu.async_copy(
                    x_hbm.at[pl.ds(r0 + _NBUF * _CHUNK, _CHUNK)], ins[b], sis[b]
                )

        return 0

    lax.fori_loop(0, _NGROUP, group, 0)
    for b in range(_NBUF):
        pltpu.make_async_copy(
            outs[b], out_hbm.at[pl.ds(row0, _CHUNK)], sos[b]
        ).wait()


_deinterleave = functools.partial(
    pl.kernel,
    out_type=jax.ShapeDtypeStruct((_ROWS, _OUT_COLS), jnp.float32),
    mesh=plsc.VectorSubcoreMesh(core_axis_name="c", subcore_axis_name="s"),
    scratch_types=(
        [pltpu.VMEM((_CHUNK, _COLS), jnp.float32)] * _NBUF
        + [pltpu.VMEM((_CHUNK, _OUT_COLS), jnp.float32)] * _NBUF
        + [pltpu.SemaphoreType.DMA] * (2 * _NBUF)
    ),
    compiler_params=pltpu.CompilerParams(
        needs_layout_passes=False,
        disable_bounds_checks=True,
        disable_semaphore_checks=True,
    ),
)(_body)


def kernel(x):
    return _deinterleave(x)


# speedup vs baseline: 1.0147x; 1.0147x over previous
import functools

import jax
import jax.numpy as jnp
from jax import lax
from jax.experimental import pallas as pl
from jax.experimental.pallas import tpu as pltpu
from jax.experimental.pallas import tpu_sc as plsc

_ROWS, _COLS = 16384, 512
_OUT_COLS = _COLS // 2
_NC, _NS, _L = 2, 16, 16
_NW = _NC * _NS
_RPW = _ROWS // _NW
_CHUNK = 32
_NBUF = 4
_NCHUNK = _RPW // _CHUNK
_NGROUP = _NCHUNK // _NBUF
_JPR = _OUT_COLS // _L


def _body(
    x_hbm, out_hbm,
    in0, in1, in2, in3, out0, out1, out2, out3,
    si0, si1, si2, si3, so0, so1, so2, so3,
):
    wid = lax.axis_index("s") * _NC + lax.axis_index("c")
    lane = lax.iota(jnp.int32, 16)
    row0 = wid * _RPW
    ins, outs = (in0, in1, in2, in3), (out0, out1, out2, out3)
    sis, sos = (si0, si1, si2, si3), (so0, so1, so2, so3)

    for b in range(_NBUF):
        pltpu.async_copy(
            x_hbm.at[pl.ds(row0 + b * _CHUNK, _CHUNK)], ins[b], sis[b]
        )

    def group(g, _):
        for b in range(_NBUF):
            c = _NBUF * g + b
            r0 = row0 + c * _CHUNK
            pltpu.make_async_copy(
                x_hbm.at[pl.ds(r0, _CHUNK)], ins[b], sis[b]
            ).wait()

            @pl.when(g > 0)
            def _():
                pltpu.make_async_copy(
                    outs[b], out_hbm.at[pl.ds(r0, _CHUNK)], sos[b]
                ).wait()

            ib, ob = ins[b], outs[b]

            @plsc.parallel_loop(0, _CHUNK, step=1, unroll=1)
            def _(r):
                rowv = jnp.full((16,), r, jnp.int32)
                for j in range(_JPR):
                    col = 2 * _L * j + 2 * lane
                    v = plsc.load_gather(ib, [rowv, col])
                    ob[r, pl.ds(_L * j, _L)] = v
            pltpu.async_copy(
                outs[b], out_hbm.at[pl.ds(r0, _CHUNK)], sos[b]
            )

            @pl.when(g < _NGROUP - 1)
            def _():
                pltpu.async_copy(
                    x_hbm.at[pl.ds(r0 + _NBUF * _CHUNK, _CHUNK)], ins[b], sis[b]
                )

        return 0

    lax.fori_loop(0, _NGROUP, group, 0)
    for b in range(_NBUF):
        pltpu.make_async_copy(
            outs[b], out_hbm.at[pl.ds(row0, _CHUNK)], sos[b]
        ).wait()


_deinterleave = functools.partial(
    pl.kernel,
    out_type=jax.ShapeDtypeStruct((_ROWS, _OUT_COLS), jnp.float32),
    mesh=plsc.VectorSubcoreMesh(core_axis_name="c", subcore_axis_name="s"),
    scratch_types=(
        [pltpu.VMEM((_CHUNK, _COLS), jnp.float32)] * _NBUF
        + [pltpu.VMEM((_CHUNK, _OUT_COLS), jnp.float32)] * _NBUF
        + [pltpu.SemaphoreType.DMA] * (2 * _NBUF)
    ),
    compiler_params=pltpu.CompilerParams(
        needs_layout_passes=False,
        disable_bounds_checks=True,
        disable_semaphore_checks=True,
    ),
)(_body)


def kernel(x):
    return _deinterleave(x)
